# Initial kernel scaffold; baseline (speedup 1.0000x reference)
import functools

import jax
import jax.numpy as jnp
from jax import lax
from jax.experimental import pallas as pl
from jax.experimental.pallas import tpu as pltpu
from jax.experimental.pallas import tpu_sc as plsc

DIM = 128
BATCH = 16384
NC = 2
NS = 16
L = 16
NW = NC * NS
BPW = BATCH // NW
CH = 128
NCH = BPW // CH


def _w2v_body(center_hbm, context_hbm, inw_hbm, outw_hbm, o_hbm,
              cidx_v, xidx_v, v_v, u_v, o_v, sem_v, sem_u):
    wid = lax.axis_index("s") * NC + lax.axis_index("c")

    pltpu.sync_copy(center_hbm.at[wid], cidx_v)
    pltpu.sync_copy(context_hbm.at[wid], xidx_v)

    cp_v = [None] * NCH
    cp_u = [None] * NCH
    cp_v[0] = pltpu.async_copy(inw_hbm.at[cidx_v.at[0]], v_v.at[0], sem_v)
    cp_u[0] = pltpu.async_copy(outw_hbm.at[xidx_v.at[0]], u_v.at[0], sem_u)

    for c in range(NCH):
        buf = c % 2
        if c + 1 < NCH:
            nbuf = (c + 1) % 2
            cp_v[c + 1] = pltpu.async_copy(
                inw_hbm.at[cidx_v.at[c + 1]], v_v.at[nbuf], sem_v)
            cp_u[c + 1] = pltpu.async_copy(
                outw_hbm.at[xidx_v.at[c + 1]], u_v.at[nbuf], sem_u)
        cp_v[c].wait()
        cp_u[c].wait()

        def row_body(r, _, buf=buf, c=c):
            acc = v_v[buf, r, pl.ds(0, L)] * u_v[buf, r, pl.ds(0, L)]
            for k in range(1, DIM // L):
                acc = acc + v_v[buf, r, pl.ds(k * L, L)] * u_v[buf, r, pl.ds(k * L, L)]
            o_v[c * CH + r] = jnp.sum(acc)
            return 0

        lax.fori_loop(0, CH, row_body, 0, unroll=2)

    pltpu.sync_copy(o_v, o_hbm.at[pl.ds(wid * BPW, BPW)])


def kernel(center_ids, context_ids, in_embed_w, out_embed_w):
    center_r = center_ids.reshape(NW, NCH, CH).astype(jnp.int32)
    context_r = context_ids.reshape(NW, NCH, CH).astype(jnp.int32)

    mesh = plsc.VectorSubcoreMesh(core_axis_name="c", subcore_axis_name="s")
    run = pl.kernel(
        _w2v_body,
        mesh=mesh,
        out_type=jax.ShapeDtypeStruct((BATCH,), jnp.float32),
        scratch_types=[
            pltpu.VMEM((NCH, CH), jnp.int32),
            pltpu.VMEM((NCH, CH), jnp.int32),
            pltpu.VMEM((2, CH, DIM), jnp.float32),
            pltpu.VMEM((2, CH, DIM), jnp.float32),
            pltpu.VMEM((BPW,), jnp.float32),
            pltpu.SemaphoreType.DMA,
            pltpu.SemaphoreType.DMA,
        ],
    )
    return run(center_r, context_r, in_embed_w, out_embed_w)


# speedup vs baseline: 1.0919x; 1.0919x over previous
import functools

import jax
import jax.numpy as jnp
from jax import lax
from jax.experimental import pallas as pl
from jax.experimental.pallas import tpu as pltpu
from jax.experimental.pallas import tpu_sc as plsc

DIM = 128
BATCH = 16384
NC = 2
NS = 16
L = 16
NW = NC * NS
BPW = BATCH // NW
CH = 128
NCH = BPW // CH


def _w2v_body(center_hbm, context_hbm, inw_hbm, outw_hbm, o_hbm,
              cidx_v, xidx_v, v_v, u_v, o_v, sem_v, sem_u):
    wid = lax.axis_index("s") * NC + lax.axis_index("c")

    pltpu.sync_copy(center_hbm.at[wid], cidx_v)
    pltpu.sync_copy(context_hbm.at[wid], xidx_v)

    cp_v = [None] * NCH
    cp_u = [None] * NCH
    cp_v[0] = pltpu.async_copy(inw_hbm.at[cidx_v.at[0]], v_v.at[0], sem_v)
    cp_u[0] = pltpu.async_copy(outw_hbm.at[xidx_v.at[0]], u_v.at[0], sem_u)

    for c in range(NCH):
        buf = c % 2
        if c + 1 < NCH:
            nbuf = (c + 1) % 2
            cp_v[c + 1] = pltpu.async_copy(
                inw_hbm.at[cidx_v.at[c + 1]], v_v.at[nbuf], sem_v)
            cp_u[c + 1] = pltpu.async_copy(
                outw_hbm.at[xidx_v.at[c + 1]], u_v.at[nbuf], sem_u)
        cp_v[c].wait()
        cp_u[c].wait()

        lane = lax.iota(jnp.int32, L)

        def group_body(g, _, buf=buf, c=c):
            res = jnp.zeros((L,), jnp.float32)
            for j in range(L):
                r = g * L + j
                acc = v_v[buf, r, pl.ds(0, L)] * u_v[buf, r, pl.ds(0, L)]
                for k in range(1, DIM // L):
                    acc = acc + v_v[buf, r, pl.ds(k * L, L)] * u_v[buf, r, pl.ds(k * L, L)]
                for s in (8, 4, 2, 1):
                    acc = acc + acc.at[lane ^ s].get(mode="promise_in_bounds")
                res = jnp.where(lane == j, acc, res)
            o_v[pl.ds(c * CH + g * L, L)] = res
            return 0

        lax.fori_loop(0, CH // L, group_body, 0)

    pltpu.sync_copy(o_v, o_hbm.at[pl.ds(wid * BPW, BPW)])


def kernel(center_ids, context_ids, in_embed_w, out_embed_w):
    center_r = center_ids.reshape(NW, NCH, CH).astype(jnp.int32)
    context_r = context_ids.reshape(NW, NCH, CH).astype(jnp.int32)

    mesh = plsc.VectorSubcoreMesh(core_axis_name="c", subcore_axis_name="s")
    run = pl.kernel(
        _w2v_body,
        mesh=mesh,
        out_type=jax.ShapeDtypeStruct((BATCH,), jnp.float32),
        scratch_types=[
            pltpu.VMEM((NCH, CH), jnp.int32),
            pltpu.VMEM((NCH, CH), jnp.int32),
            pltpu.VMEM((2, CH, DIM), jnp.float32),
            pltpu.VMEM((2, CH, DIM), jnp.float32),
            pltpu.VMEM((BPW,), jnp.float32),
            pltpu.SemaphoreType.DMA,
            pltpu.SemaphoreType.DMA,
        ],
    )
    return run(center_r, context_r, in_embed_w, out_embed_w)


# speedup vs baseline: 1.5567x; 1.4257x over previous
import functools

import jax
import jax.numpy as jnp
from jax import lax
from jax.experimental import pallas as pl
from jax.experimental.pallas import tpu as pltpu
from jax.experimental.pallas import tpu_sc as plsc

DIM = 128
BATCH = 16384
NC = 2
NS = 16
L = 16
NW = NC * NS
BPW = BATCH // NW
CH = 128
NCH = BPW // CH


def _w2v_body(center_hbm, context_hbm, inw_hbm, outw_hbm, o_hbm,
              cidx_v, xidx_v, v_v, u_v, o_v, sem_v, sem_u):
    wid = lax.axis_index("s") * NC + lax.axis_index("c")

    pltpu.sync_copy(center_hbm.at[wid], cidx_v)
    pltpu.sync_copy(context_hbm.at[wid], xidx_v)

    cp_v = [None] * NCH
    cp_u = [None] * NCH
    cp_v[0] = pltpu.async_copy(inw_hbm.at[cidx_v.at[0]], v_v.at[0], sem_v)
    cp_u[0] = pltpu.async_copy(outw_hbm.at[xidx_v.at[0]], u_v.at[0], sem_u)

    for c in range(NCH):
        buf = c % 2
        if c + 1 < NCH:
            nbuf = (c + 1) % 2
            cp_v[c + 1] = pltpu.async_copy(
                inw_hbm.at[cidx_v.at[c + 1]], v_v.at[nbuf], sem_v)
            cp_u[c + 1] = pltpu.async_copy(
                outw_hbm.at[xidx_v.at[c + 1]], u_v.at[nbuf], sem_u)
        cp_v[c].wait()
        cp_u[c].wait()

        lane = lax.iota(jnp.int32, L)
        RSUB = 4

        def group_body(g, _, buf=buf, c=c):
            def sub_body(s, res, buf=buf):
                for jj in range(RSUB):
                    j = s * RSUB + jj
                    r = g * L + j
                    acc = v_v[buf, r, pl.ds(0, L)] * u_v[buf, r, pl.ds(0, L)]
                    for k in range(1, DIM // L):
                        acc = acc + v_v[buf, r, pl.ds(k * L, L)] * u_v[buf, r, pl.ds(k * L, L)]
                    for sh in (8, 4, 2, 1):
                        acc = acc + acc.at[lane ^ sh].get(mode="promise_in_bounds")
                    res = jnp.where(lane == j, acc, res)
                return res

            res = lax.fori_loop(0, L // RSUB, sub_body, jnp.zeros((L,), jnp.float32))
            o_v[pl.ds(c * CH + g * L, L)] = res
            return 0

        lax.fori_loop(0, CH // L, group_body, 0)

    pltpu.sync_copy(o_v, o_hbm.at[pl.ds(wid * BPW, BPW)])


def kernel(center_ids, context_ids, in_embed_w, out_embed_w):
    center_r = center_ids.reshape(NW, NCH, CH).astype(jnp.int32)
    context_r = context_ids.reshape(NW, NCH, CH).astype(jnp.int32)

    mesh = plsc.VectorSubcoreMesh(core_axis_name="c", subcore_axis_name="s")
    run = pl.kernel(
        _w2v_body,
        mesh=mesh,
        out_type=jax.ShapeDtypeStruct((BATCH,), jnp.float32),
        scratch_types=[
            pltpu.VMEM((NCH, CH), jnp.int32),
            pltpu.VMEM((NCH, CH), jnp.int32),
            pltpu.VMEM((2, CH, DIM), jnp.float32),
            pltpu.VMEM((2, CH, DIM), jnp.float32),
            pltpu.VMEM((BPW,), jnp.float32),
            pltpu.SemaphoreType.DMA,
            pltpu.SemaphoreType.DMA,
        ],
    )
    return run(center_r, context_r, in_embed_w, out_embed_w)


# speedup vs baseline: 1.6168x; 1.0386x over previous
import jax
import jax.numpy as jnp
from jax import lax
from jax.experimental import pallas as pl
from jax.experimental.pallas import tpu as pltpu
from jax.experimental.pallas import tpu_sc as plsc

DIM = 128
BATCH = 16384
NC = 2
NS = 16
L = 16
NW = NC * NS
BPW = BATCH // NW
CH = 128
NCH = BPW // CH
NBUF = 3


def _w2v_body(center_hbm, context_hbm, inw_hbm, outw_hbm, o_hbm,
              cidx_v, xidx_v, v_v, u_v, o_v, sem_v, sem_u, sem_i, sem_o):
    wid = lax.axis_index("s") * NC + lax.axis_index("c")

    ci = pltpu.async_copy(center_hbm.at[wid], cidx_v, sem_i)
    xi = pltpu.async_copy(context_hbm.at[wid], xidx_v, sem_i)
    ci.wait()
    xi.wait()

    cp_v = [None] * NCH
    cp_u = [None] * NCH
    for c in range(NBUF - 1):
        cp_v[c] = pltpu.async_copy(inw_hbm.at[cidx_v.at[c]], v_v.at[c % NBUF], sem_v)
        cp_u[c] = pltpu.async_copy(outw_hbm.at[xidx_v.at[c]], u_v.at[c % NBUF], sem_u)

    out_cp = [None] * NCH
    lane = lax.iota(jnp.int32, L)
    RSUB = 4

    for c in range(NCH):
        buf = c % NBUF
        n = c + NBUF - 1
        if n < NCH:
            cp_v[n] = pltpu.async_copy(inw_hbm.at[cidx_v.at[n]], v_v.at[n % NBUF], sem_v)
            cp_u[n] = pltpu.async_copy(outw_hbm.at[xidx_v.at[n]], u_v.at[n % NBUF], sem_u)
        cp_v[c].wait()
        cp_u[c].wait()

        def group_body(g, _, buf=buf, c=c):
            def sub_body(s, res, buf=buf):
                for jj in range(RSUB):
                    j = s * RSUB + jj
                    r = g * L + j
                    acc = v_v[buf, r, pl.ds(0, L)] * u_v[buf, r, pl.ds(0, L)]
                    for k in range(1, DIM // L):
                        acc = acc + v_v[buf, r, pl.ds(k * L, L)] * u_v[buf, r, pl.ds(k * L, L)]
                    for sh in (8, 4, 2, 1):
                        acc = acc + acc.at[lane ^ sh].get(mode="promise_in_bounds")
                    res = jnp.where(lane == j, acc, res)
                return res

            res = lax.fori_loop(0, L // RSUB, sub_body, jnp.zeros((L,), jnp.float32))
            o_v[pl.ds(c * CH + g * L, L)] = res
            return 0

        lax.fori_loop(0, CH // L, group_body, 0)
        out_cp[c] = pltpu.async_copy(
            o_v.at[pl.ds(c * CH, CH)], o_hbm.at[pl.ds(wid * BPW + c * CH, CH)], sem_o)

    for c in range(NCH):
        out_cp[c].wait()


def kernel(center_ids, context_ids, in_embed_w, out_embed_w):
    center_r = center_ids.reshape(NW, NCH, CH).astype(jnp.int32)
    context_r = context_ids.reshape(NW, NCH, CH).astype(jnp.int32)

    mesh = plsc.VectorSubcoreMesh(core_axis_name="c", subcore_axis_name="s")
    run = pl.kernel(
        _w2v_body,
        mesh=mesh,
        out_type=jax.ShapeDtypeStruct((BATCH,), jnp.float32),
        scratch_types=[
            pltpu.VMEM((NCH, CH), jnp.int32),
            pltpu.VMEM((NCH, CH), jnp.int32),
            pltpu.VMEM((NBUF, CH, DIM), jnp.float32),
            pltpu.VMEM((NBUF, CH, DIM), jnp.float32),
            pltpu.VMEM((BPW,), jnp.float32),
            pltpu.SemaphoreType.DMA,
            pltpu.SemaphoreType.DMA,
            pltpu.SemaphoreType.DMA,
            pltpu.SemaphoreType.DMA,
        ],
    )
    return run(center_r, context_r, in_embed_w, out_embed_w)
